# baseline (speedup 1.0000x reference)
import functools

import jax
import jax.numpy as jnp
from jax import lax
from jax.experimental import pallas as pl
from jax.experimental.pallas import tpu as pltpu
from jax.experimental.pallas import tpu_sc as plsc

_EMB = 32
_NUM_CORES = 2
_NUM_SUBCORES = 16
_NW = _NUM_CORES * _NUM_SUBCORES
_BLK = 1024
_B = 16384
_F = 26
_GPF = _B // _BLK
_NBLK = _F * _GPF // _NW


def _make_gather():
    n = _B * _F
    per_w = n // _NW
    mesh = plsc.VectorSubcoreMesh(core_axis_name="c", subcore_axis_name="s")

    @functools.partial(
        pl.kernel,
        mesh=mesh,
        out_type=jax.ShapeDtypeStruct((_F * _EMB * _B // 128, 128), jnp.float32),
        scratch_types=[
            pltpu.VMEM((per_w,), jnp.int32),
            [pltpu.VMEM((_BLK, _EMB), jnp.float32) for _ in range(2)],
            pltpu.VMEM((_BLK * _EMB // 128, 128), jnp.float32),
            [pltpu.SemaphoreType.DMA for _ in range(2)],
            [pltpu.SemaphoreType.DMA for _ in range(2)],
        ],
        compiler_params=pltpu.CompilerParams(
            use_tc_tiling_on_sc=False, needs_layout_passes=False
        ),
    )
    def emb_kernel(idx_hbm, table_hbm, out_hbm, idx_v, gbufs, tbuf, gsems, tsems):
        wid = lax.axis_index("s") * _NUM_CORES + lax.axis_index("c")
        base = wid * per_w
        iota = lax.iota(jnp.int32, 16)
        rvec0 = (iota // 8) * 64 + iota % 8

        def out_row0(k):
            blk = wid * _NBLK + k
            f = blk // _GPF
            g = blk % _GPF
            return f * (_EMB // 8) * (_B // 128) * 8 + g * 64

        def gather(k, b):
            pltpu.async_copy(
                table_hbm.at[idx_v.at[pl.ds(k * _BLK, _BLK)]],
                gbufs[b],
                gsems[b],
            )

        def wait_gather(k, b):
            pltpu.make_async_copy(
                table_hbm.at[idx_v.at[pl.ds(k * _BLK, _BLK)]],
                gbufs[b],
                gsems[b],
            ).wait()

        def out_dmas(k, wait):
            r0 = out_row0(k)
            for dt in range(4):
                cp = pltpu.make_async_copy(
                    tbuf.at[pl.ds(dt * 64, 64), :],
                    out_hbm.at[pl.ds(r0 + dt * 1024, 64), :],
                    tsems[dt % 2],
                )
                if wait:
                    cp.wait()
                else:
                    cp.start()

        def shuffle(b):
            gbuf = gbufs[b]

            def body(i, carry):
                btl = i // 16
                bc0 = (i % 16) * 8
                rv0 = rvec0 + btl * 8
                rv16 = rv0 + 128
                for j in range(8):
                    row = i * 8 + j
                    col = jnp.zeros((16,), jnp.int32) + (bc0 + j)
                    plsc.store_scatter(
                        tbuf, [rv0, col], gbuf[row, pl.ds(0, 16)]
                    )
                    plsc.store_scatter(
                        tbuf, [rv16, col], gbuf[row, pl.ds(16, 16)]
                    )
                return carry

            lax.fori_loop(0, _BLK // 8, body, 0)

        pltpu.sync_copy(idx_hbm.at[pl.ds(base, per_w)], idx_v)
        gather(0, 0)
        gather(1, 1)
        for k in range(_NBLK):
            b = k % 2
            wait_gather(k, b)
            if k >= 1:
                out_dmas(k - 1, wait=True)
            shuffle(b)
            if k + 2 < _NBLK:
                gather(k + 2, b)
            out_dmas(k, wait=False)
        out_dmas(_NBLK - 1, wait=True)

    return emb_kernel


def kernel(holder, table):
    b, f = holder.shape
    idx = holder.T.reshape(-1).astype(jnp.int32)
    out128 = _make_gather()(idx, table)
    out = (
        out128.reshape(f, _EMB // 8, b // 128, 8, 128)
        .transpose(2, 4, 0, 1, 3)
        .reshape(b, f, _EMB)
    )
    return out
